# baseline (speedup 1.0000x reference)
import functools

import jax
import jax.numpy as jnp
from jax import lax
from jax.experimental import pallas as pl
from jax.experimental.pallas import tpu as pltpu
from jax.experimental.pallas import tpu_sc as plsc

N = 10000
E = 320000
D = 128
NG = 64
NCLS = 10

NC, NS = 2, 16
NW = NC * NS
NP = 10112
RPT = NP // NS

BLK = 400
GRID = N // BLK


def _seg_body(with_cnt, nring, sdepth, chunk, nsteps, off_fn, ntail,
              feat, src1d, dst1d, zrows, out, *rest):
    if with_cnt:
        cnt32 = rest[0]
        rest = rest[1:]
    acc = rest[0]
    bufs = rest[1:]
    if with_cnt:
        cntloc = bufs[0]
        bufs = bufs[1:]
    srcv = bufs[0:nring]
    dstv = bufs[nring:2 * nring]
    rows = bufs[2 * nring:3 * nring]
    gsem = bufs[3 * nring:4 * nring]
    ssem = bufs[4 * nring:5 * nring]
    isem = bufs[5 * nring:6 * nring]

    cid = lax.axis_index("c")
    sid = lax.axis_index("s")
    wid = cid * NS + sid

    pltpu.sync_copy(zrows, acc.at[pl.ds(sid * RPT, RPT)])

    if with_cnt:
        zeros16 = jnp.zeros((16,), jnp.float32)

        def zcnt(i, _):
            cntloc[pl.ds(i * 16, 16)] = zeros16
            return 0
        lax.fori_loop(0, N // 16, zcnt, 0)
        ones16 = jnp.ones((16,), jnp.float32)

    plsc.subcore_barrier()

    def fire_idx(c, b):
        off = off_fn(wid, c)
        pltpu.async_copy(src1d.at[pl.ds(off, chunk)], srcv[b], isem[b])
        pltpu.async_copy(dst1d.at[pl.ds(off, chunk)], dstv[b], isem[b])

    def wait_idx(c, b):
        off = off_fn(wid, c)
        pltpu.make_async_copy(src1d.at[pl.ds(off, chunk)],
                              srcv[b], isem[b]).wait()
        pltpu.make_async_copy(dst1d.at[pl.ds(off, chunk)],
                              dstv[b], isem[b]).wait()

    def wait_scatter(b):
        pltpu.make_async_copy(rows[b], acc.at[dstv[b]], ssem[b]).wait()

    off0 = off_fn(wid, 0)
    pltpu.sync_copy(src1d.at[pl.ds(off0, chunk)], srcv[0])
    pltpu.sync_copy(dst1d.at[pl.ds(off0, chunk)], dstv[0])
    pltpu.async_copy(feat.at[srcv[0]], rows[0], gsem[0])
    fire_idx(1, 1)

    def ring_step(cc, _):
        for k in range(nring):
            c = cc * nring + k

            @pl.when((c >= sdepth) & (c < nsteps + sdepth))
            def _():
                wait_scatter((k + nring - sdepth) % nring)

            @pl.when(c < nsteps)
            def _():
                @pl.when(c + 1 < nsteps)
                def _():
                    wait_idx(c + 1, (k + 1) % nring)
                    pltpu.async_copy(feat.at[srcv[(k + 1) % nring]],
                                     rows[(k + 1) % nring],
                                     gsem[(k + 1) % nring])

                @pl.when(c + 2 < nsteps)
                def _():
                    fire_idx(c + 2, (k + 2) % nring)
                pltpu.make_async_copy(
                    feat.at[srcv[k]], rows[k], gsem[k]).wait()
                pltpu.async_copy(rows[k], acc.at[dstv[k]], ssem[k],
                                 add=True)
                if with_cnt:
                    for g in range(chunk // 16):
                        idx = dstv[k][pl.ds(g * 16, 16)]
                        plsc.addupdate_scatter(cntloc, [idx], ones16)
        return 0
    lax.fori_loop(0, (nsteps + sdepth + nring - 1) // nring, ring_step, 0)

    if ntail:
        @pl.when(wid < ntail)
        def _():
            off = (nsteps * NW + wid) * chunk
            pltpu.sync_copy(src1d.at[pl.ds(off, chunk)], srcv[0])
            pltpu.sync_copy(dst1d.at[pl.ds(off, chunk)], dstv[0])
            pltpu.async_copy(feat.at[srcv[0]], rows[0], gsem[0]).wait()
            pltpu.sync_copy(rows[0], acc.at[dstv[0]], add=True)

    plsc.subcore_barrier()

    row0 = sid * RPT
    pltpu.sync_copy(acc.at[pl.ds(row0, RPT)],
                    out.at[pl.ds(cid * NP + row0, RPT)])
    if with_cnt:
        pltpu.sync_copy(cntloc, cnt32.at[wid])


def _make_segsum(with_cnt):
    if with_cnt:
        nring, sdepth, chunk = 3, 1, 80
        ept = E // NW
        nsteps = ept // chunk
        ntail = 0

        def off_fn(wid, c):
            return wid * ept + c * chunk
    else:
        nring, sdepth, chunk = 3, 1, 128
        nsteps = E // (chunk * NW)
        ntail = E // chunk - nsteps * NW

        def off_fn(wid, c):
            return (wid + c * NW) * chunk

    mesh = plsc.VectorSubcoreMesh(core_axis_name="c", subcore_axis_name="s")
    out_type = [jax.ShapeDtypeStruct((NC * NP, D), jnp.float32)]
    scratch = [pltpu.VMEM_SHARED((NP, D), jnp.float32)]
    if with_cnt:
        out_type.append(jax.ShapeDtypeStruct((NW, N), jnp.float32))
        scratch.append(pltpu.VMEM((N,), jnp.float32))
    scratch += (
        [pltpu.VMEM((chunk,), jnp.int32)] * nring
        + [pltpu.VMEM((chunk,), jnp.int32)] * nring
        + [pltpu.VMEM((chunk, D), jnp.float32)] * nring
        + [pltpu.SemaphoreType.DMA] * (3 * nring)
    )
    params = None
    if with_cnt:
        params = pltpu.CompilerParams(use_tc_tiling_on_sc=False,
                                      needs_layout_passes=False)
    return pl.kernel(
        functools.partial(_seg_body, with_cnt, nring, sdepth, chunk,
                          nsteps, off_fn, ntail),
        out_type=tuple(out_type),
        mesh=mesh,
        compiler_params=params,
        scratch_types=scratch,
    )


_segsum_cnt = _make_segsum(True)
_segsum = _make_segsum(False)


def _dense_body(parts, cnt32, x, wl, wr, b, out):
    cnt = jnp.sum(cnt32[:, 0, 0, :], axis=0)[:, None]
    agg = (parts[0] + parts[1]) / jnp.maximum(cnt, 1.0)
    h = agg @ wl[...] + x[...] @ wr[...] + b[...]
    out[...] = jnp.maximum(h, 0.0)


def _final_body(parts, cnt32, h1, batch, wl, wr, b, wfc, bfc, out,
                gacc, cacc):
    i = pl.program_id(0)

    @pl.when(i == 0)
    def _():
        gacc[...] = jnp.zeros_like(gacc)
        cacc[...] = jnp.zeros_like(cacc)

    cnt = jnp.sum(cnt32[:, 0, 0, :], axis=0)[:, None]
    agg = (parts[0] + parts[1]) / jnp.maximum(cnt, 1.0)
    h2 = jnp.maximum(agg @ wl[...] + h1[...] @ wr[...] + b[...], 0.0)

    bt = batch[0, 0, :]
    P = (lax.broadcasted_iota(jnp.int32, (NG, BLK), 0)
         == bt[None, :]).astype(jnp.float32)
    gacc[...] += P @ h2
    cacc[...] += P @ jnp.ones((BLK, D), jnp.float32)

    @pl.when(i == GRID - 1)
    def _():
        g = gacc[...] / jnp.maximum(cacc[...], 1.0)
        out[...] = g @ wfc[...] + bfc[...]


def _dense(parts, cnt32, x, wl, wr, b):
    return pl.pallas_call(
        _dense_body,
        grid=(GRID,),
        in_specs=[
            pl.BlockSpec((NC, BLK, D), lambda i: (0, i, 0)),
            pl.BlockSpec((NW, 1, 1, BLK), lambda i: (0, i, 0, 0)),
            pl.BlockSpec((BLK, D), lambda i: (i, 0)),
            pl.BlockSpec((D, D), lambda i: (0, 0)),
            pl.BlockSpec((D, D), lambda i: (0, 0)),
            pl.BlockSpec((1, D), lambda i: (0, 0)),
        ],
        out_specs=pl.BlockSpec((BLK, D), lambda i: (i, 0)),
        out_shape=jax.ShapeDtypeStruct((N, D), jnp.float32),
    )(parts, cnt32, x, wl, wr, b)


def _final(parts, cnt32, h1, batch3, wl, wr, b, wfc, bfc):
    return pl.pallas_call(
        _final_body,
        grid=(GRID,),
        in_specs=[
            pl.BlockSpec((NC, BLK, D), lambda i: (0, i, 0)),
            pl.BlockSpec((NW, 1, 1, BLK), lambda i: (0, i, 0, 0)),
            pl.BlockSpec((BLK, D), lambda i: (i, 0)),
            pl.BlockSpec((1, 1, BLK), lambda i: (i, 0, 0)),
            pl.BlockSpec((D, D), lambda i: (0, 0)),
            pl.BlockSpec((D, D), lambda i: (0, 0)),
            pl.BlockSpec((1, D), lambda i: (0, 0)),
            pl.BlockSpec((D, NCLS), lambda i: (0, 0)),
            pl.BlockSpec((1, NCLS), lambda i: (0, 0)),
        ],
        out_specs=pl.BlockSpec((NG, NCLS), lambda i: (0, 0)),
        out_shape=jax.ShapeDtypeStruct((NG, NCLS), jnp.float32),
        scratch_shapes=[
            pltpu.VMEM((NG, D), jnp.float32),
            pltpu.VMEM((NG, D), jnp.float32),
        ],
    )(parts, cnt32, h1, batch3, wl, wr, b, wfc, bfc)


def kernel(x, edge_index, batch, W1l, W1r, b1, W2l, W2r, b2, Wfc, bfc):
    src = edge_index[0].astype(jnp.int32)
    dst = edge_index[1].astype(jnp.int32)
    bt3 = batch.astype(jnp.int32).reshape(GRID, 1, BLK)

    zrows = jnp.zeros((RPT, D), jnp.float32)
    parts1, cnt32 = _segsum_cnt(x, src, dst, zrows)
    cnt32 = cnt32.reshape(NW, GRID, 1, BLK)
    parts1 = parts1.reshape(NC, NP, D)
    h1 = _dense(parts1, cnt32, x, W1l, W1r, b1.reshape(1, D))
    (parts2,) = _segsum(h1, src, dst, zrows)
    parts2 = parts2.reshape(NC, NP, D)
    out = _final(parts2, cnt32, h1, bt3, W2l, W2r, b2.reshape(1, D),
                 Wfc, bfc.reshape(1, NCLS))
    return out


# speedup vs baseline: 1.0276x; 1.0276x over previous
import functools

import jax
import jax.numpy as jnp
from jax import lax
from jax.experimental import pallas as pl
from jax.experimental.pallas import tpu as pltpu
from jax.experimental.pallas import tpu_sc as plsc

N = 10000
E = 320000
D = 128
NG = 64
NCLS = 10

NC, NS = 2, 16
NW = NC * NS
NP = 10112
RPT = NP // NS

BLK = 400
GRID = N // BLK


def _seg_body(with_cnt, nring, sdepth, chunk, nsteps, off_fn, ntail,
              feat, src1d, dst1d, zrows, out, *rest):
    if with_cnt:
        cnt32 = rest[0]
        rest = rest[1:]
    acc = rest[0]
    bufs = rest[1:]
    if with_cnt:
        cntloc = bufs[0]
        bufs = bufs[1:]
    srcv = bufs[0:nring]
    dstv = bufs[nring:2 * nring]
    rows = bufs[2 * nring:3 * nring]
    gsem = bufs[3 * nring:4 * nring]
    ssem = bufs[4 * nring:5 * nring]
    isem = bufs[5 * nring:6 * nring]

    cid = lax.axis_index("c")
    sid = lax.axis_index("s")
    wid = cid * NS + sid

    pltpu.sync_copy(zrows, acc.at[pl.ds(sid * RPT, RPT)])

    if with_cnt:
        zeros16 = jnp.zeros((16,), jnp.float32)

        def zcnt(i, _):
            cntloc[pl.ds(i * 16, 16)] = zeros16
            return 0
        lax.fori_loop(0, N // 16, zcnt, 0)
        ones16 = jnp.ones((16,), jnp.float32)

    plsc.subcore_barrier()

    def fire_idx(c, b):
        off = off_fn(wid, c)
        pltpu.async_copy(src1d.at[pl.ds(off, chunk)], srcv[b], isem[b])
        pltpu.async_copy(dst1d.at[pl.ds(off, chunk)], dstv[b], isem[b])

    def wait_idx(c, b):
        off = off_fn(wid, c)
        pltpu.make_async_copy(src1d.at[pl.ds(off, chunk)],
                              srcv[b], isem[b]).wait()
        pltpu.make_async_copy(dst1d.at[pl.ds(off, chunk)],
                              dstv[b], isem[b]).wait()

    def wait_scatter(b):
        pltpu.make_async_copy(rows[b], acc.at[dstv[b]], ssem[b]).wait()

    off0 = off_fn(wid, 0)
    pltpu.sync_copy(src1d.at[pl.ds(off0, chunk)], srcv[0])
    pltpu.sync_copy(dst1d.at[pl.ds(off0, chunk)], dstv[0])
    pltpu.async_copy(feat.at[srcv[0]], rows[0], gsem[0])
    fire_idx(1, 1)

    def ring_step(cc, _):
        for k in range(nring):
            c = cc * nring + k

            @pl.when((c >= sdepth) & (c < nsteps + sdepth))
            def _():
                wait_scatter((k + nring - sdepth) % nring)

            @pl.when(c < nsteps)
            def _():
                @pl.when(c + 1 < nsteps)
                def _():
                    wait_idx(c + 1, (k + 1) % nring)
                    pltpu.async_copy(feat.at[srcv[(k + 1) % nring]],
                                     rows[(k + 1) % nring],
                                     gsem[(k + 1) % nring])

                @pl.when(c + 2 < nsteps)
                def _():
                    fire_idx(c + 2, (k + 2) % nring)
                pltpu.make_async_copy(
                    feat.at[srcv[k]], rows[k], gsem[k]).wait()
                pltpu.async_copy(rows[k], acc.at[dstv[k]], ssem[k],
                                 add=True)
                if with_cnt:
                    for g in range(chunk // 16):
                        idx = dstv[k][pl.ds(g * 16, 16)]
                        plsc.addupdate_scatter(cntloc, [idx], ones16)
        return 0
    lax.fori_loop(0, (nsteps + sdepth + nring - 1) // nring, ring_step, 0)

    if ntail:
        @pl.when(wid < ntail)
        def _():
            off = (nsteps * NW + wid) * chunk
            pltpu.sync_copy(src1d.at[pl.ds(off, chunk)], srcv[0])
            pltpu.sync_copy(dst1d.at[pl.ds(off, chunk)], dstv[0])
            pltpu.async_copy(feat.at[srcv[0]], rows[0], gsem[0]).wait()
            pltpu.sync_copy(rows[0], acc.at[dstv[0]], add=True)

    plsc.subcore_barrier()

    row0 = sid * RPT
    pltpu.sync_copy(acc.at[pl.ds(row0, RPT)],
                    out.at[pl.ds(cid * NP + row0, RPT)])
    if with_cnt:
        pltpu.sync_copy(cntloc, cnt32.at[wid])


def _make_segsum(with_cnt):
    if with_cnt:
        nring, sdepth, chunk = 3, 1, 80
        ept = E // NW
        nsteps = ept // chunk
        ntail = 0

        def off_fn(wid, c):
            return wid * ept + c * chunk
    else:
        nring, sdepth, chunk = 4, 2, 80
        ept = E // NW
        nsteps = ept // chunk
        ntail = 0

        def off_fn(wid, c):
            return wid * ept + c * chunk

    mesh = plsc.VectorSubcoreMesh(core_axis_name="c", subcore_axis_name="s")
    out_type = [jax.ShapeDtypeStruct((NC * NP, D), jnp.float32)]
    scratch = [pltpu.VMEM_SHARED((NP, D), jnp.float32)]
    if with_cnt:
        out_type.append(jax.ShapeDtypeStruct((NW, N), jnp.float32))
        scratch.append(pltpu.VMEM((N,), jnp.float32))
    scratch += (
        [pltpu.VMEM((chunk,), jnp.int32)] * nring
        + [pltpu.VMEM((chunk,), jnp.int32)] * nring
        + [pltpu.VMEM((chunk, D), jnp.float32)] * nring
        + [pltpu.SemaphoreType.DMA] * (3 * nring)
    )
    params = None
    if with_cnt:
        params = pltpu.CompilerParams(use_tc_tiling_on_sc=False,
                                      needs_layout_passes=False)
    return pl.kernel(
        functools.partial(_seg_body, with_cnt, nring, sdepth, chunk,
                          nsteps, off_fn, ntail),
        out_type=tuple(out_type),
        mesh=mesh,
        compiler_params=params,
        scratch_types=scratch,
    )


_segsum_cnt = _make_segsum(True)
_segsum = _make_segsum(False)


def _dense_body(parts, cnt32, x, wl, wr, b, out):
    cnt = jnp.sum(cnt32[:, 0, 0, :], axis=0)[:, None]
    agg = (parts[0] + parts[1]) / jnp.maximum(cnt, 1.0)
    h = agg @ wl[...] + x[...] @ wr[...] + b[...]
    out[...] = jnp.maximum(h, 0.0)


def _final_body(parts, cnt32, h1, batch, wl, wr, b, wfc, bfc, out,
                gacc, cacc):
    i = pl.program_id(0)

    @pl.when(i == 0)
    def _():
        gacc[...] = jnp.zeros_like(gacc)
        cacc[...] = jnp.zeros_like(cacc)

    cnt = jnp.sum(cnt32[:, 0, 0, :], axis=0)[:, None]
    agg = (parts[0] + parts[1]) / jnp.maximum(cnt, 1.0)
    h2 = jnp.maximum(agg @ wl[...] + h1[...] @ wr[...] + b[...], 0.0)

    bt = batch[0, 0, :]
    P = (lax.broadcasted_iota(jnp.int32, (NG, BLK), 0)
         == bt[None, :]).astype(jnp.float32)
    gacc[...] += P @ h2
    cacc[...] += P @ jnp.ones((BLK, D), jnp.float32)

    @pl.when(i == GRID - 1)
    def _():
        g = gacc[...] / jnp.maximum(cacc[...], 1.0)
        out[...] = g @ wfc[...] + bfc[...]


def _dense(parts, cnt32, x, wl, wr, b):
    return pl.pallas_call(
        _dense_body,
        grid=(GRID,),
        in_specs=[
            pl.BlockSpec((NC, BLK, D), lambda i: (0, i, 0)),
            pl.BlockSpec((NW, 1, 1, BLK), lambda i: (0, i, 0, 0)),
            pl.BlockSpec((BLK, D), lambda i: (i, 0)),
            pl.BlockSpec((D, D), lambda i: (0, 0)),
            pl.BlockSpec((D, D), lambda i: (0, 0)),
            pl.BlockSpec((1, D), lambda i: (0, 0)),
        ],
        out_specs=pl.BlockSpec((BLK, D), lambda i: (i, 0)),
        out_shape=jax.ShapeDtypeStruct((N, D), jnp.float32),
    )(parts, cnt32, x, wl, wr, b)


def _final(parts, cnt32, h1, batch3, wl, wr, b, wfc, bfc):
    return pl.pallas_call(
        _final_body,
        grid=(GRID,),
        in_specs=[
            pl.BlockSpec((NC, BLK, D), lambda i: (0, i, 0)),
            pl.BlockSpec((NW, 1, 1, BLK), lambda i: (0, i, 0, 0)),
            pl.BlockSpec((BLK, D), lambda i: (i, 0)),
            pl.BlockSpec((1, 1, BLK), lambda i: (i, 0, 0)),
            pl.BlockSpec((D, D), lambda i: (0, 0)),
            pl.BlockSpec((D, D), lambda i: (0, 0)),
            pl.BlockSpec((1, D), lambda i: (0, 0)),
            pl.BlockSpec((D, NCLS), lambda i: (0, 0)),
            pl.BlockSpec((1, NCLS), lambda i: (0, 0)),
        ],
        out_specs=pl.BlockSpec((NG, NCLS), lambda i: (0, 0)),
        out_shape=jax.ShapeDtypeStruct((NG, NCLS), jnp.float32),
        scratch_shapes=[
            pltpu.VMEM((NG, D), jnp.float32),
            pltpu.VMEM((NG, D), jnp.float32),
        ],
    )(parts, cnt32, h1, batch3, wl, wr, b, wfc, bfc)


def kernel(x, edge_index, batch, W1l, W1r, b1, W2l, W2r, b2, Wfc, bfc):
    src = edge_index[0].astype(jnp.int32)
    dst = edge_index[1].astype(jnp.int32)
    bt3 = batch.astype(jnp.int32).reshape(GRID, 1, BLK)

    zrows = jnp.zeros((RPT, D), jnp.float32)
    parts1, cnt32 = _segsum_cnt(x, src, dst, zrows)
    cnt32 = cnt32.reshape(NW, GRID, 1, BLK)
    parts1 = parts1.reshape(NC, NP, D)
    h1 = _dense(parts1, cnt32, x, W1l, W1r, b1.reshape(1, D))
    (parts2,) = _segsum(h1, src, dst, zrows)
    parts2 = parts2.reshape(NC, NP, D)
    out = _final(parts2, cnt32, h1, bt3, W2l, W2r, b2.reshape(1, D),
                 Wfc, bfc.reshape(1, NCLS))
    return out


# speedup vs baseline: 1.0490x; 1.0208x over previous
import functools

import jax
import jax.numpy as jnp
from jax import lax
from jax.experimental import pallas as pl
from jax.experimental.pallas import tpu as pltpu
from jax.experimental.pallas import tpu_sc as plsc

N = 10000
E = 320000
D = 128
NG = 64
NCLS = 10

NC, NS = 2, 16
NW = NC * NS
NP = 10112
RPT = NP // NS

BLK = 400
GRID = N // BLK


def _seg_body(with_cnt, nring, sdepth, chunk, nsteps, off_fn, ntail,
              feat, src1d, dst1d, zrows, out, *rest):
    if with_cnt:
        cnt32 = rest[0]
        rest = rest[1:]
    acc = rest[0]
    bufs = rest[1:]
    if with_cnt:
        cntloc = bufs[0]
        bufs = bufs[1:]
    srcv = bufs[0:nring]
    dstv = bufs[nring:2 * nring]
    rows = bufs[2 * nring:3 * nring]
    gsem = bufs[3 * nring:4 * nring]
    ssem = bufs[4 * nring:5 * nring]
    isem = bufs[5 * nring:6 * nring]

    cid = lax.axis_index("c")
    sid = lax.axis_index("s")
    wid = cid * NS + sid

    pltpu.sync_copy(zrows, acc.at[pl.ds(sid * RPT, RPT)])

    if with_cnt:
        zeros16 = jnp.zeros((16,), jnp.float32)

        def zcnt(i, _):
            cntloc[pl.ds(i * 16, 16)] = zeros16
            return 0
        lax.fori_loop(0, N // 16, zcnt, 0)
        ones16 = jnp.ones((16,), jnp.float32)

    plsc.subcore_barrier()

    def fire_idx(c, b):
        off = off_fn(wid, c)
        pltpu.async_copy(src1d.at[pl.ds(off, chunk)], srcv[b], isem[b])
        pltpu.async_copy(dst1d.at[pl.ds(off, chunk)], dstv[b], isem[b])

    def wait_idx(c, b):
        off = off_fn(wid, c)
        pltpu.make_async_copy(src1d.at[pl.ds(off, chunk)],
                              srcv[b], isem[b]).wait()
        pltpu.make_async_copy(dst1d.at[pl.ds(off, chunk)],
                              dstv[b], isem[b]).wait()

    def wait_scatter(b):
        pltpu.make_async_copy(rows[b], acc.at[dstv[b]], ssem[b]).wait()

    off0 = off_fn(wid, 0)
    pltpu.sync_copy(src1d.at[pl.ds(off0, chunk)], srcv[0])
    pltpu.sync_copy(dst1d.at[pl.ds(off0, chunk)], dstv[0])
    pltpu.async_copy(feat.at[srcv[0]], rows[0], gsem[0])
    fire_idx(1, 1)

    def ring_step(cc, _):
        for k in range(nring):
            c = cc * nring + k

            @pl.when((c >= sdepth) & (c < nsteps + sdepth))
            def _():
                wait_scatter((k + nring - sdepth) % nring)

            @pl.when(c < nsteps)
            def _():
                @pl.when(c + 1 < nsteps)
                def _():
                    wait_idx(c + 1, (k + 1) % nring)
                    pltpu.async_copy(feat.at[srcv[(k + 1) % nring]],
                                     rows[(k + 1) % nring],
                                     gsem[(k + 1) % nring])

                @pl.when(c + 2 < nsteps)
                def _():
                    fire_idx(c + 2, (k + 2) % nring)
                pltpu.make_async_copy(
                    feat.at[srcv[k]], rows[k], gsem[k]).wait()
                pltpu.async_copy(rows[k], acc.at[dstv[k]], ssem[k],
                                 add=True)
                if with_cnt:
                    for g in range(chunk // 16):
                        idx = dstv[k][pl.ds(g * 16, 16)]
                        plsc.addupdate_scatter(cntloc, [idx], ones16)
        return 0
    lax.fori_loop(0, (nsteps + sdepth + nring - 1) // nring, ring_step, 0)

    if ntail:
        @pl.when(wid < ntail)
        def _():
            off = (nsteps * NW + wid) * chunk
            pltpu.sync_copy(src1d.at[pl.ds(off, chunk)], srcv[0])
            pltpu.sync_copy(dst1d.at[pl.ds(off, chunk)], dstv[0])
            pltpu.async_copy(feat.at[srcv[0]], rows[0], gsem[0]).wait()
            pltpu.sync_copy(rows[0], acc.at[dstv[0]], add=True)
            if with_cnt:
                for g in range(chunk // 16):
                    idx = dstv[0][pl.ds(g * 16, 16)]
                    plsc.addupdate_scatter(cntloc, [idx], ones16)

    plsc.subcore_barrier()

    row0 = sid * RPT
    pltpu.sync_copy(acc.at[pl.ds(row0, RPT)],
                    out.at[pl.ds(cid * NP + row0, RPT)])
    if with_cnt:
        pltpu.sync_copy(cntloc, cnt32.at[wid])


def _make_segsum(with_cnt):
    if with_cnt:
        nring, sdepth, chunk = 4, 2, 64
        nsteps = E // (chunk * NW)
        ntail = E // chunk - nsteps * NW

        def off_fn(wid, c):
            return (wid + c * NW) * chunk
    else:
        nring, sdepth, chunk = 4, 2, 80
        ept = E // NW
        nsteps = ept // chunk
        ntail = 0

        def off_fn(wid, c):
            return wid * ept + c * chunk

    mesh = plsc.VectorSubcoreMesh(core_axis_name="c", subcore_axis_name="s")
    out_type = [jax.ShapeDtypeStruct((NC * NP, D), jnp.float32)]
    scratch = [pltpu.VMEM_SHARED((NP, D), jnp.float32)]
    if with_cnt:
        out_type.append(jax.ShapeDtypeStruct((NW, N), jnp.float32))
        scratch.append(pltpu.VMEM((N,), jnp.float32))
    scratch += (
        [pltpu.VMEM((chunk,), jnp.int32)] * nring
        + [pltpu.VMEM((chunk,), jnp.int32)] * nring
        + [pltpu.VMEM((chunk, D), jnp.float32)] * nring
        + [pltpu.SemaphoreType.DMA] * (3 * nring)
    )
    params = None
    if with_cnt:
        params = pltpu.CompilerParams(use_tc_tiling_on_sc=False,
                                      needs_layout_passes=False)
    return pl.kernel(
        functools.partial(_seg_body, with_cnt, nring, sdepth, chunk,
                          nsteps, off_fn, ntail),
        out_type=tuple(out_type),
        mesh=mesh,
        compiler_params=params,
        scratch_types=scratch,
    )


_segsum_cnt = _make_segsum(True)
_segsum = _make_segsum(False)


def _dense_body(parts, cnt32, x, wl, wr, b, out):
    cnt = jnp.sum(cnt32[:, 0, 0, :], axis=0)[:, None]
    agg = (parts[0] + parts[1]) / jnp.maximum(cnt, 1.0)
    h = agg @ wl[...] + x[...] @ wr[...] + b[...]
    out[...] = jnp.maximum(h, 0.0)


def _final_body(parts, cnt32, h1, batch, wl, wr, b, wfc, bfc, out,
                gacc, cacc):
    i = pl.program_id(0)

    @pl.when(i == 0)
    def _():
        gacc[...] = jnp.zeros_like(gacc)
        cacc[...] = jnp.zeros_like(cacc)

    cnt = jnp.sum(cnt32[:, 0, 0, :], axis=0)[:, None]
    agg = (parts[0] + parts[1]) / jnp.maximum(cnt, 1.0)
    h2 = jnp.maximum(agg @ wl[...] + h1[...] @ wr[...] + b[...], 0.0)

    bt = batch[0, 0, :]
    P = (lax.broadcasted_iota(jnp.int32, (NG, BLK), 0)
         == bt[None, :]).astype(jnp.float32)
    gacc[...] += P @ h2
    cacc[...] += P @ jnp.ones((BLK, D), jnp.float32)

    @pl.when(i == GRID - 1)
    def _():
        g = gacc[...] / jnp.maximum(cacc[...], 1.0)
        out[...] = g @ wfc[...] + bfc[...]


def _dense(parts, cnt32, x, wl, wr, b):
    return pl.pallas_call(
        _dense_body,
        grid=(GRID,),
        in_specs=[
            pl.BlockSpec((NC, BLK, D), lambda i: (0, i, 0)),
            pl.BlockSpec((NW, 1, 1, BLK), lambda i: (0, i, 0, 0)),
            pl.BlockSpec((BLK, D), lambda i: (i, 0)),
            pl.BlockSpec((D, D), lambda i: (0, 0)),
            pl.BlockSpec((D, D), lambda i: (0, 0)),
            pl.BlockSpec((1, D), lambda i: (0, 0)),
        ],
        out_specs=pl.BlockSpec((BLK, D), lambda i: (i, 0)),
        out_shape=jax.ShapeDtypeStruct((N, D), jnp.float32),
    )(parts, cnt32, x, wl, wr, b)


def _final(parts, cnt32, h1, batch3, wl, wr, b, wfc, bfc):
    return pl.pallas_call(
        _final_body,
        grid=(GRID,),
        in_specs=[
            pl.BlockSpec((NC, BLK, D), lambda i: (0, i, 0)),
            pl.BlockSpec((NW, 1, 1, BLK), lambda i: (0, i, 0, 0)),
            pl.BlockSpec((BLK, D), lambda i: (i, 0)),
            pl.BlockSpec((1, 1, BLK), lambda i: (i, 0, 0)),
            pl.BlockSpec((D, D), lambda i: (0, 0)),
            pl.BlockSpec((D, D), lambda i: (0, 0)),
            pl.BlockSpec((1, D), lambda i: (0, 0)),
            pl.BlockSpec((D, NCLS), lambda i: (0, 0)),
            pl.BlockSpec((1, NCLS), lambda i: (0, 0)),
        ],
        out_specs=pl.BlockSpec((NG, NCLS), lambda i: (0, 0)),
        out_shape=jax.ShapeDtypeStruct((NG, NCLS), jnp.float32),
        scratch_shapes=[
            pltpu.VMEM((NG, D), jnp.float32),
            pltpu.VMEM((NG, D), jnp.float32),
        ],
    )(parts, cnt32, h1, batch3, wl, wr, b, wfc, bfc)


def kernel(x, edge_index, batch, W1l, W1r, b1, W2l, W2r, b2, Wfc, bfc):
    src = edge_index[0].astype(jnp.int32)
    dst = edge_index[1].astype(jnp.int32)
    bt3 = batch.astype(jnp.int32).reshape(GRID, 1, BLK)

    zrows = jnp.zeros((RPT, D), jnp.float32)
    parts1, cnt32 = _segsum_cnt(x, src, dst, zrows)
    cnt32 = cnt32.reshape(NW, GRID, 1, BLK)
    parts1 = parts1.reshape(NC, NP, D)
    h1 = _dense(parts1, cnt32, x, W1l, W1r, b1.reshape(1, D))
    (parts2,) = _segsum(h1, src, dst, zrows)
    parts2 = parts2.reshape(NC, NP, D)
    out = _final(parts2, cnt32, h1, bt3, W2l, W2r, b2.reshape(1, D),
                 Wfc, bfc.reshape(1, NCLS))
    return out
